# Initial kernel scaffold; baseline (speedup 1.0000x reference)
import jax
import jax.numpy as jnp
from jax.experimental import pallas as pl


def kernel(x_rumor, x_stance, sadj, fadj, s1_W1, s1_b1, s1_W2, s1_b2, s2_W1, s2_b1, s2_W2, s2_b2, c_W1, c_b1, c_W2, c_b2, mlp1_W, mlp1_b, mlp2_W, mlp2_b):
    raise NotImplementedError("write your pallas kernel here")


# speedup vs baseline: 1.8186x; 1.8186x over previous
import jax
import jax.numpy as jnp
from jax.experimental import pallas as pl


def _xw_body(x_ref, w_ref, o_ref):
    o_ref[...] = jnp.dot(
        x_ref[...].astype(jnp.bfloat16),
        w_ref[...].astype(jnp.bfloat16),
        preferred_element_type=jnp.float32,
    ).astype(jnp.bfloat16)


def _xw(x, w, bm=2000):
    n, k = x.shape
    bm = min(bm, n)
    m = w.shape[1]
    return pl.pallas_call(
        _xw_body,
        grid=(n // bm,),
        in_specs=[
            pl.BlockSpec((bm, k), lambda i: (i, 0)),
            pl.BlockSpec((k, m), lambda i: (0, 0)),
        ],
        out_specs=pl.BlockSpec((bm, m), lambda i: (i, 0)),
        out_shape=jax.ShapeDtypeStruct((n, m), jnp.bfloat16),
    )(x, w)


def _layer1_body(adj_ref, y_ref, b1_ref, w2_ref, o_ref):
    h = jnp.dot(
        adj_ref[...].astype(jnp.bfloat16),
        y_ref[...],
        preferred_element_type=jnp.float32,
    )
    h = jnp.maximum(h + b1_ref[...], 0.0).astype(jnp.bfloat16)
    o_ref[...] = jnp.dot(
        h, w2_ref[...], preferred_element_type=jnp.float32
    ).astype(jnp.bfloat16)


def _layer1(adj, y, b1, w2, bm=200):
    n = adj.shape[0]
    bm = min(bm, n)
    c1 = y.shape[1]
    c2 = w2.shape[1]
    return pl.pallas_call(
        _layer1_body,
        grid=(n // bm,),
        in_specs=[
            pl.BlockSpec((bm, n), lambda i: (i, 0)),
            pl.BlockSpec((n, c1), lambda i: (0, 0)),
            pl.BlockSpec((1, c1), lambda i: (0, 0)),
            pl.BlockSpec((c1, c2), lambda i: (0, 0)),
        ],
        out_specs=pl.BlockSpec((bm, c2), lambda i: (i, 0)),
        out_shape=jax.ShapeDtypeStruct((n, c2), jnp.bfloat16),
    )(adj, y, b1, w2)


def _layer2_body(adj_ref, z_ref, b2_ref, mw_ref, mb_ref, o_ref):
    s = (
        jnp.dot(
            adj_ref[...].astype(jnp.bfloat16),
            z_ref[...],
            preferred_element_type=jnp.float32,
        )
        + b2_ref[...]
    )
    logits = (
        jnp.dot(
            s.astype(jnp.bfloat16),
            mw_ref[...].astype(jnp.bfloat16),
            preferred_element_type=jnp.float32,
        )
        + mb_ref[...]
    )
    mx = jnp.max(logits, axis=1, keepdims=True)
    sh = logits - mx
    o_ref[...] = sh - jnp.log(jnp.sum(jnp.exp(sh), axis=1, keepdims=True))


def _layer2(adj, z, b2, mw, mb, bm=200):
    n = adj.shape[0]
    bm = min(bm, n)
    c = z.shape[1]
    p = mw.shape[1]
    return pl.pallas_call(
        _layer2_body,
        grid=(n // bm,),
        in_specs=[
            pl.BlockSpec((bm, n), lambda i: (i, 0)),
            pl.BlockSpec((n, c), lambda i: (0, 0)),
            pl.BlockSpec((1, c), lambda i: (0, 0)),
            pl.BlockSpec((c, p), lambda i: (0, 0)),
            pl.BlockSpec((1, p), lambda i: (0, 0)),
        ],
        out_specs=pl.BlockSpec((bm, p), lambda i: (i, 0)),
        out_shape=jax.ShapeDtypeStruct((n, p), jnp.float32),
    )(adj, z, b2, mw, mb)


def _branch(x, adj, sW1, sb1, sW2, sb2, cW1, cb1, cW2, cb2, mW, mb):
    h2 = sW2.shape[1]
    w1cat = jnp.concatenate([sW1, cW1], axis=1)
    b1cat = jnp.concatenate([sb1, cb1]).reshape(1, -1)
    w2blk = jnp.zeros((sW2.shape[0] + cW2.shape[0], 2 * h2), jnp.bfloat16)
    w2blk = w2blk.at[: sW2.shape[0], :h2].set(sW2.astype(jnp.bfloat16))
    w2blk = w2blk.at[sW2.shape[0] :, h2:].set(cW2.astype(jnp.bfloat16))
    b2cat = jnp.concatenate([sb2, cb2]).reshape(1, -1)
    nout = mW.shape[1]
    mwp = jnp.zeros((mW.shape[0], 8), jnp.float32).at[:, :nout].set(mW)
    mbp = jnp.full((1, 8), -1e30, jnp.float32).at[0, :nout].set(mb)

    y = _xw(x, w1cat)
    z = _layer1(adj, y, b1cat, w2blk)
    out = _layer2(adj, z, b2cat, mwp, mbp)
    return out[:, :nout]


def kernel(x_rumor, x_stance, sadj, fadj,
           s1_W1, s1_b1, s1_W2, s1_b2,
           s2_W1, s2_b1, s2_W2, s2_b2,
           c_W1, c_b1, c_W2, c_b2,
           mlp1_W, mlp1_b, mlp2_W, mlp2_b):
    output1 = _branch(x_rumor, sadj, s1_W1, s1_b1, s1_W2, s1_b2,
                      c_W1, c_b1, c_W2, c_b2, mlp1_W, mlp1_b)
    output2 = _branch(x_stance, fadj, s2_W1, s2_b1, s2_W2, s2_b2,
                      c_W1, c_b1, c_W2, c_b2, mlp2_W, mlp2_b)
    return (output1, output2)


# speedup vs baseline: 1.8948x; 1.0419x over previous
import jax
import jax.numpy as jnp
from jax.experimental import pallas as pl


def _xw_body(x_ref, w_ref, o_ref):
    o_ref[...] = jnp.dot(
        x_ref[...].astype(jnp.bfloat16),
        w_ref[...].astype(jnp.bfloat16),
        preferred_element_type=jnp.float32,
    ).astype(jnp.bfloat16)


def _xw(x, w, bm=2000):
    n, k = x.shape
    bm = min(bm, n)
    m = w.shape[1]
    return pl.pallas_call(
        _xw_body,
        grid=(n // bm,),
        in_specs=[
            pl.BlockSpec((bm, k), lambda i: (i, 0)),
            pl.BlockSpec((k, m), lambda i: (0, 0)),
        ],
        out_specs=pl.BlockSpec((bm, m), lambda i: (i, 0)),
        out_shape=jax.ShapeDtypeStruct((n, m), jnp.bfloat16),
    )(x, w)


def _layer1_body(adj_ref, y_ref, b1_ref, w2_ref, o_ref):
    h = jnp.dot(
        adj_ref[...].astype(jnp.bfloat16),
        y_ref[...],
        preferred_element_type=jnp.float32,
    )
    h = jnp.maximum(h + b1_ref[...], 0.0).astype(jnp.bfloat16)
    o_ref[...] = jnp.dot(
        h, w2_ref[...], preferred_element_type=jnp.float32
    ).astype(jnp.bfloat16)


def _layer1(adj, y, b1, w2, bm=400):
    n = adj.shape[0]
    bm = min(bm, n)
    c1 = y.shape[1]
    c2 = w2.shape[1]
    return pl.pallas_call(
        _layer1_body,
        grid=(n // bm,),
        in_specs=[
            pl.BlockSpec((bm, n), lambda i: (i, 0)),
            pl.BlockSpec((n, c1), lambda i: (0, 0)),
            pl.BlockSpec((1, c1), lambda i: (0, 0)),
            pl.BlockSpec((c1, c2), lambda i: (0, 0)),
        ],
        out_specs=pl.BlockSpec((bm, c2), lambda i: (i, 0)),
        out_shape=jax.ShapeDtypeStruct((n, c2), jnp.bfloat16),
    )(adj, y, b1, w2)


def _layer2_body(adj_ref, z_ref, b2_ref, mw_ref, mb_ref, o_ref):
    s = (
        jnp.dot(
            adj_ref[...].astype(jnp.bfloat16),
            z_ref[...],
            preferred_element_type=jnp.float32,
        )
        + b2_ref[...]
    )
    logits = (
        jnp.dot(
            s.astype(jnp.bfloat16),
            mw_ref[...].astype(jnp.bfloat16),
            preferred_element_type=jnp.float32,
        )
        + mb_ref[...]
    )
    mx = jnp.max(logits, axis=1, keepdims=True)
    sh = logits - mx
    o_ref[...] = sh - jnp.log(jnp.sum(jnp.exp(sh), axis=1, keepdims=True))


def _layer2(adj, z, b2, mw, mb, bm=400):
    n = adj.shape[0]
    bm = min(bm, n)
    c = z.shape[1]
    p = mw.shape[1]
    return pl.pallas_call(
        _layer2_body,
        grid=(n // bm,),
        in_specs=[
            pl.BlockSpec((bm, n), lambda i: (i, 0)),
            pl.BlockSpec((n, c), lambda i: (0, 0)),
            pl.BlockSpec((1, c), lambda i: (0, 0)),
            pl.BlockSpec((c, p), lambda i: (0, 0)),
            pl.BlockSpec((1, p), lambda i: (0, 0)),
        ],
        out_specs=pl.BlockSpec((bm, p), lambda i: (i, 0)),
        out_shape=jax.ShapeDtypeStruct((n, p), jnp.float32),
    )(adj, z, b2, mw, mb)


def _branch(x, adj, sW1, sb1, sW2, sb2, cW1, cb1, cW2, cb2, mW, mb):
    h2 = sW2.shape[1]
    w1cat = jnp.concatenate([sW1, cW1], axis=1)
    b1cat = jnp.concatenate([sb1, cb1]).reshape(1, -1)
    w2blk = jnp.zeros((sW2.shape[0] + cW2.shape[0], 2 * h2), jnp.bfloat16)
    w2blk = w2blk.at[: sW2.shape[0], :h2].set(sW2.astype(jnp.bfloat16))
    w2blk = w2blk.at[sW2.shape[0] :, h2:].set(cW2.astype(jnp.bfloat16))
    b2cat = jnp.concatenate([sb2, cb2]).reshape(1, -1)
    nout = mW.shape[1]
    mwp = jnp.zeros((mW.shape[0], 8), jnp.float32).at[:, :nout].set(mW)
    mbp = jnp.full((1, 8), -1e30, jnp.float32).at[0, :nout].set(mb)

    y = _xw(x, w1cat)
    z = _layer1(adj, y, b1cat, w2blk)
    out = _layer2(adj, z, b2cat, mwp, mbp)
    return out[:, :nout]


def kernel(x_rumor, x_stance, sadj, fadj,
           s1_W1, s1_b1, s1_W2, s1_b2,
           s2_W1, s2_b1, s2_W2, s2_b2,
           c_W1, c_b1, c_W2, c_b2,
           mlp1_W, mlp1_b, mlp2_W, mlp2_b):
    output1 = _branch(x_rumor, sadj, s1_W1, s1_b1, s1_W2, s1_b2,
                      c_W1, c_b1, c_W2, c_b2, mlp1_W, mlp1_b)
    output2 = _branch(x_stance, fadj, s2_W1, s2_b1, s2_W2, s2_b2,
                      c_W1, c_b1, c_W2, c_b2, mlp2_W, mlp2_b)
    return (output1, output2)
